# baseline (speedup 1.0000x reference)
import jax
import jax.numpy as jnp
from jax import lax
from jax.experimental import pallas as pl
from jax.experimental.pallas import tpu as pltpu, tpu_sc as plsc

B = 4096
L = 50
D = 64
DPAD = 128
LANES = 16
NC, NS = 2, 16
NW = NC * NS
GROUPS = B // (NW * LANES)
IDS_PER_GROUP = LANES * L


LH = L // 2
WPAD = 64


def _sc_body(ids_hbm, w_hbm, table_hbm, out_hbm, idx0, idx1, w0, w1,
             rows_a, rows_b, out0, out1, sem_a, sem_b, sem_s, sem_o):
    cid = lax.axis_index("c")
    sid = lax.axis_index("s")
    wid = sid * NC + cid

    idx_bufs = (idx0, idx1)
    w_bufs = (w0, w1)
    out_bufs = (out0, out1)

    def stage(g, sync):
        gidx = wid * GROUPS + g
        b0 = gidx * LANES
        p = g % 2
        if sync:
            pltpu.sync_copy(ids_hbm.at[gidx], idx_bufs[p])
            pltpu.sync_copy(w_hbm.at[pl.ds(b0, LANES)], w_bufs[p])
            return ()
        return (
            pltpu.async_copy(ids_hbm.at[gidx], idx_bufs[p], sem_s),
            pltpu.async_copy(w_hbm.at[pl.ds(b0, LANES)], w_bufs[p], sem_s),
        )

    def fire(g, half, rows_buf, sem):
        idx_v = idx_bufs[g % 2]
        return [
            pltpu.async_copy(
                table_hbm.at[idx_v.at[half * 4 + r]],
                rows_buf.at[pl.ds(r * 100, 100), :],
                sem,
            )
            for r in range(4)
        ]

    def accumulate(g, half, rows_buf):
        w_v = w_bufs[g % 2]
        out_v = out_bufs[g % 2]

        def b_body(bl, carry):
            b = half * (LANES // 2) + bl
            chunks = {c: w_v[b, pl.ds(16 * c, 16)] for c in range(4)}
            accs = [jnp.zeros((LANES,), jnp.float32) for _ in range(D // LANES)]
            for j in range(L):
                lane = jnp.full((LANES,), j % 16, jnp.int32)
                wb = lax.gather(
                    chunks[j // 16], lane[:, None],
                    dimension_numbers=lax.GatherDimensionNumbers(
                        offset_dims=(), collapsed_slice_dims=(0,),
                        start_index_map=(0,)),
                    slice_sizes=(1,),
                    mode=lax.GatherScatterMode.PROMISE_IN_BOUNDS)
                row = bl * L + j
                for db in range(D // LANES):
                    vals = rows_buf[row, pl.ds(db * 16, 16)]
                    accs[db] = accs[db] + wb * vals
            for db in range(D // LANES):
                out_v[b, pl.ds(db * 16, 16)] = accs[db]
            return carry

        lax.fori_loop(0, LANES // 2, b_body, 0)

    stage(0, sync=True)
    copies_a = fire(0, 0, rows_a, sem_a)
    copies_b = fire(0, 1, rows_b, sem_b)
    out_copies = [None, None]
    for g in range(GROUPS):
        staging = stage(g + 1, sync=False) if g + 1 < GROUPS else ()
        for c in copies_a:
            c.wait()
        if out_copies[g % 2] is not None:
            out_copies[g % 2].wait()
            out_copies[g % 2] = None
        accumulate(g, 0, rows_a)
        for c in copies_b:
            c.wait()
        if g + 1 < GROUPS:
            for c in staging:
                c.wait()
            copies_a = fire(g + 1, 0, rows_a, sem_a)
        accumulate(g, 1, rows_b)
        if g + 1 < GROUPS:
            copies_b = fire(g + 1, 1, rows_b, sem_b)
        b0 = (wid * GROUPS + g) * LANES
        out_copies[g % 2] = pltpu.async_copy(
            out_bufs[g % 2], out_hbm.at[pl.ds(b0, LANES)], sem_o)
    for c in out_copies:
        if c is not None:
            c.wait()


_sc_kernel = pl.kernel(
    _sc_body,
    out_type=jax.ShapeDtypeStruct((B, D), jnp.float32),
    mesh=plsc.VectorSubcoreMesh(core_axis_name="c", subcore_axis_name="s"),
    scratch_types=[
        pltpu.VMEM((8, 100), jnp.int32),
        pltpu.VMEM((8, 100), jnp.int32),
        pltpu.VMEM((LANES, WPAD), jnp.float32),
        pltpu.VMEM((LANES, WPAD), jnp.float32),
        pltpu.VMEM((IDS_PER_GROUP // 2, DPAD), jnp.float32),
        pltpu.VMEM((IDS_PER_GROUP // 2, DPAD), jnp.float32),
        pltpu.VMEM((LANES, D), jnp.float32),
        pltpu.VMEM((LANES, D), jnp.float32),
        pltpu.SemaphoreType.DMA,
        pltpu.SemaphoreType.DMA,
        pltpu.SemaphoreType.DMA,
        pltpu.SemaphoreType.DMA,
    ],
    compiler_params=pltpu.CompilerParams(
        use_tc_tiling_on_sc=True, needs_layout_passes=False
    ),
)


def kernel(sp_ids, sp_weights, embeddings):
    emb_pad = jnp.pad(embeddings, ((0, 0), (0, DPAD - D)))
    ids_r = sp_ids.reshape(B // LANES, 8, 100)
    w_pad = jnp.pad(sp_weights, ((0, 0), (0, WPAD - L)))
    return _sc_kernel(ids_r, w_pad, emb_pad)
